# baseline (speedup 1.0000x reference)
import functools

import jax
import jax.numpy as jnp
from jax import lax
from jax.experimental import pallas as pl
from jax.experimental.pallas import tpu as pltpu
import jax.experimental.pallas.tpu_sc as plsc

N = 10000
E = 320000
D = 128
DP3 = 16
N_PAD = 10240
NC = 2
NS = 16
NW = NC * NS
EPW = 10240
K = 80
NB = EPW // K
RPS = N_PAD // NS

_mesh = plsc.VectorSubcoreMesh(core_axis_name="c", subcore_axis_name="s")


def _worker_range(wid, k=K, nbmax=NB):
    base0 = wid * EPW
    base_st = jnp.minimum(base0, E - EPW)
    loff = base0 - base_st
    nb = jnp.minimum(nbmax, (E - base0) // k)
    return base_st, loff, nb


CHW = 2560


def _stage_edges(edge_hbm, base_st, ed_ch, src_all, dst_all, want_src):
    for c in range(EPW // CHW):
        pltpu.sync_copy(edge_hbm.at[:, pl.ds(base_st + c * CHW, CHW)],
                        ed_ch)

        def cp(i, _):
            o = c * CHW + i * 16
            if want_src:
                src_all[pl.ds(o, 16)] = ed_ch[0, pl.ds(i * 16, 16)]
            dst_all[pl.ds(o, 16)] = ed_ch[1, pl.ds(i * 16, 16)]
            return 0

        lax.fori_loop(0, CHW // 16, cp, 0)


@functools.partial(
    pl.kernel,
    out_type=jax.ShapeDtypeStruct((NC, N_PAD), jnp.float32),
    mesh=_mesh,
    scratch_types=[
        pltpu.VMEM((2, CHW), jnp.int32),
        pltpu.VMEM((EPW,), jnp.int32),
        pltpu.VMEM((K,), jnp.float32),
        pltpu.SemaphoreType.DMA,
        pltpu.VMEM_SHARED((N_PAD,), jnp.float32),
    ],
)
def _sc_count(edge_hbm, zeros_hbm, out_hbm, ed_ch, dst_all, ones_v, ssem,
              acc_sh):
    cid = lax.axis_index("c")
    sid = lax.axis_index("s")
    wid = sid * NC + cid
    base_st, loff, nb = _worker_range(wid)

    def fill_ones(i, _):
        ones_v[pl.ds(i * 16, 16)] = jnp.ones((16,), jnp.float32)
        return 0

    lax.fori_loop(0, K // 16, fill_ones, 0)
    pltpu.sync_copy(zeros_hbm.at[pl.ds(sid * RPS, RPS)],
                    acc_sh.at[pl.ds(sid * RPS, RPS)])
    _stage_edges(edge_hbm, base_st, ed_ch, None, dst_all, False)
    plsc.subcore_barrier()

    CDEPTH = 8

    def fire(j):
        pltpu.async_copy(ones_v,
                         acc_sh.at[dst_all.at[pl.ds(loff + j * K, K)]],
                         ssem, add=True)

    def wait_one(j):
        pltpu.make_async_copy(ones_v,
                              acc_sh.at[dst_all.at[pl.ds(loff + j * K, K)]],
                              ssem).wait()

    def prol(j, _):
        fire(j)
        return 0

    def steady(j, _):
        wait_one(j - CDEPTH)
        fire(j)
        return 0

    def drain(j, _):
        wait_one(j)
        return 0

    ndeep = jnp.minimum(nb, CDEPTH)
    lax.fori_loop(0, ndeep, prol, 0)
    lax.fori_loop(CDEPTH, nb, steady, 0)
    lax.fori_loop(nb - ndeep, nb, drain, 0)
    plsc.subcore_barrier()
    pltpu.sync_copy(acc_sh.at[pl.ds(sid * RPS, RPS)],
                    out_hbm.at[cid, pl.ds(sid * RPS, RPS)])


KA = 40
NBA = EPW // KA
ABUF = 4


@functools.partial(
    pl.kernel,
    out_type=jax.ShapeDtypeStruct((NC, N_PAD, D), jnp.float32),
    mesh=_mesh,
    scratch_types=[
        pltpu.VMEM((2, CHW), jnp.int32),
        pltpu.VMEM((EPW,), jnp.int32),
        pltpu.VMEM((EPW,), jnp.int32),
        pltpu.VMEM((ABUF, KA, D), jnp.float32),
        pltpu.SemaphoreType.DMA((ABUF,)),
        pltpu.VMEM_SHARED((N_PAD, D), jnp.float32),
    ],
)
def _sc_agg(h_hbm, edge_hbm, zeros_hbm, out_hbm,
            ed_ch, src_all, dst_all, rows_v, gsem, acc_sh):
    cid = lax.axis_index("c")
    sid = lax.axis_index("s")
    wid = sid * NC + cid
    base_st, loff, nba = _worker_range(wid, KA, NBA)
    pltpu.sync_copy(zeros_hbm.at[pl.ds(sid * RPS, RPS)],
                    acc_sh.at[pl.ds(sid * RPS, RPS)])
    _stage_edges(edge_hbm, base_st, ed_ch, src_all, dst_all, True)
    plsc.subcore_barrier()

    def fire_gather(j, b):
        pltpu.async_copy(
            h_hbm.at[src_all.at[pl.ds(loff + j * KA, KA)]],
            rows_v.at[b], gsem.at[b])

    def drain_and_scatter(j, b):
        pltpu.make_async_copy(
            h_hbm.at[src_all.at[pl.ds(loff + j * KA, KA)]],
            rows_v.at[b], gsem.at[b]).wait()
        pltpu.sync_copy(rows_v.at[b],
                        acc_sh.at[dst_all.at[pl.ds(loff + j * KA, KA)]],
                        add=True)

    for b in range(ABUF):
        fire_gather(b, b)

    def body(g, _):
        for b in range(ABUF):
            j = g * ABUF + b
            drain_and_scatter(j, b)
            fire_gather(j + ABUF, b)
        return 0

    lax.fori_loop(0, nba // ABUF - 1, body, 0)
    for b in range(ABUF):
        drain_and_scatter(nba - ABUF + b, b)
    plsc.subcore_barrier()
    pltpu.sync_copy(acc_sh.at[pl.ds(sid * RPS, RPS)],
                    out_hbm.at[cid, pl.ds(sid * RPS, RPS)])


BN = 2000
GRID = N // BN

_row2 = lambda g: (g, 0)
_row3 = lambda g: (0, g, 0)
_full = lambda g: (0, 0)


def _t12_body(x_ref, w_ref, cnt_ref, hs_ref, dinv_ref):
    dinv = lax.rsqrt(1.0 + cnt_ref[...])
    hw = jnp.dot(x_ref[...], w_ref[...], preferred_element_type=jnp.float32)
    hs_ref[...] = hw * dinv
    dinv_ref[...] = dinv


def _tc_stage1(x, W1, cnt_col):
    return pl.pallas_call(
        _t12_body,
        grid=(GRID,),
        in_specs=[
            pl.BlockSpec((BN, D), _row2),
            pl.BlockSpec((D, D), _full),
            pl.BlockSpec((BN, 1), _row2),
        ],
        out_specs=[
            pl.BlockSpec((BN, D), _row2),
            pl.BlockSpec((BN, 1), _row2),
        ],
        out_shape=[
            jax.ShapeDtypeStruct((N, D), jnp.float32),
            jax.ShapeDtypeStruct((N, 1), jnp.float32),
        ],
    )(x, W1, cnt_col)


def _tmid_body(agg_ref, hs_ref, dinv_ref, b_ref, w_ref, out_ref):
    dinv = dinv_ref[...]
    h = dinv * (agg_ref[0] + agg_ref[1] + hs_ref[...]) + b_ref[...]
    h = jnp.maximum(h, 0.0)
    hw = jnp.dot(h, w_ref[...], preferred_element_type=jnp.float32)
    out_ref[...] = hw * dinv


def _tc_mid(agg, hs, dinv_col, b_row, W):
    return pl.pallas_call(
        _tmid_body,
        grid=(GRID,),
        in_specs=[
            pl.BlockSpec((NC, BN, D), _row3),
            pl.BlockSpec((BN, D), _row2),
            pl.BlockSpec((BN, 1), _row2),
            pl.BlockSpec((1, D), _full),
            pl.BlockSpec((D, D), _full),
        ],
        out_specs=pl.BlockSpec((BN, D), _row2),
        out_shape=jax.ShapeDtypeStruct((N, D), jnp.float32),
    )(agg, hs, dinv_col, b_row, W)


def _t4_body(agg_ref, hs_ref, dinv_ref, b_ref, out_ref):
    dinv = dinv_ref[...]
    h = dinv * (agg_ref[0] + agg_ref[1] + hs_ref[...]) + b_ref[...]
    out_ref[...] = dinv * jnp.maximum(h, 0.0)


def _tc_pre3(agg, hs2, dinv_col, b2_row):
    return pl.pallas_call(
        _t4_body,
        grid=(GRID,),
        in_specs=[
            pl.BlockSpec((NC, BN, D), _row3),
            pl.BlockSpec((BN, D), _row2),
            pl.BlockSpec((BN, 1), _row2),
            pl.BlockSpec((1, D), _full),
        ],
        out_specs=pl.BlockSpec((BN, D), _row2),
        out_shape=jax.ShapeDtypeStruct((N, D), jnp.float32),
    )(agg, hs2, dinv_col, b2_row)


def _t5_body(agg_ref, g_ref, dinv_ref, w_ref, b_ref, out_ref):
    z = dinv_ref[...] * (agg_ref[0] + agg_ref[1] + g_ref[...])
    out_ref[...] = (jnp.dot(z, w_ref[...], preferred_element_type=jnp.float32)
                    + b_ref[...])


def _tc_final(agg, g, dinv_col, W3p, b3_row):
    return pl.pallas_call(
        _t5_body,
        grid=(GRID,),
        in_specs=[
            pl.BlockSpec((NC, BN, D), _row3),
            pl.BlockSpec((BN, D), _row2),
            pl.BlockSpec((BN, 1), _row2),
            pl.BlockSpec((D, DP3), _full),
            pl.BlockSpec((1, DP3), _full),
        ],
        out_specs=pl.BlockSpec((BN, DP3), _row2),
        out_shape=jax.ShapeDtypeStruct((N, DP3), jnp.float32),
    )(agg, g, dinv_col, W3p, b3_row)


def kernel(x, edge_index, W1, b1, W2, b2, W3, b3):
    W3p = jnp.pad(W3, ((0, 0), (0, DP3 - W3.shape[1])))
    b1r = b1.reshape(1, D)
    b2r = b2.reshape(1, D)
    b3r = jnp.pad(b3, (0, DP3 - b3.shape[0])).reshape(1, DP3)
    z1 = jnp.zeros((N_PAD,), jnp.float32)
    z128 = jnp.zeros((N_PAD, D), jnp.float32)

    cnt_parts = _sc_count(edge_index, z1)
    cnt_col = (cnt_parts[0] + cnt_parts[1])[:N].reshape(N, 1)

    hs1, dinv_col = _tc_stage1(x, W1, cnt_col)

    agg1 = _sc_agg(hs1, edge_index, z128)
    hs2 = _tc_mid(agg1, hs1, dinv_col, b1r, W2)

    agg2 = _sc_agg(hs2, edge_index, z128)
    g = _tc_pre3(agg2, hs2, dinv_col, b2r)

    agg3 = _sc_agg(g, edge_index, z128)
    out16 = _tc_final(agg3, g, dinv_col, W3p, b3r)

    return out16[:, :2]


# speedup vs baseline: 1.0373x; 1.0373x over previous
import functools

import jax
import jax.numpy as jnp
from jax import lax
from jax.experimental import pallas as pl
from jax.experimental.pallas import tpu as pltpu
import jax.experimental.pallas.tpu_sc as plsc

N = 10000
E = 320000
D = 128
DP3 = 16
N_PAD = 10240
NC = 2
NS = 16
NW = NC * NS
EPW = 10240
K = 80
NB = EPW // K
RPS = N_PAD // NS

_mesh = plsc.VectorSubcoreMesh(core_axis_name="c", subcore_axis_name="s")


def _worker_range(wid, k=K, nbmax=NB):
    base0 = wid * EPW
    base_st = jnp.minimum(base0, E - EPW)
    loff = base0 - base_st
    nb = jnp.minimum(nbmax, (E - base0) // k)
    return base_st, loff, nb


@functools.partial(
    pl.kernel,
    out_type=jax.ShapeDtypeStruct((NC, N_PAD), jnp.float32),
    mesh=_mesh,
    scratch_types=[
        pltpu.VMEM((EPW,), jnp.int32),
        pltpu.VMEM((K,), jnp.float32),
        pltpu.SemaphoreType.DMA,
        pltpu.VMEM_SHARED((N_PAD,), jnp.float32),
    ],
)
def _sc_count(dst_hbm, zeros_hbm, out_hbm, dst_all, ones_v, ssem, acc_sh):
    cid = lax.axis_index("c")
    sid = lax.axis_index("s")
    wid = sid * NC + cid
    base_st, loff, nb = _worker_range(wid)

    def fill_ones(i, _):
        ones_v[pl.ds(i * 16, 16)] = jnp.ones((16,), jnp.float32)
        return 0

    lax.fori_loop(0, K // 16, fill_ones, 0)
    pltpu.sync_copy(zeros_hbm.at[pl.ds(sid * RPS, RPS)],
                    acc_sh.at[pl.ds(sid * RPS, RPS)])
    pltpu.sync_copy(dst_hbm.at[pl.ds(base_st, EPW)], dst_all)
    plsc.subcore_barrier()

    CDEPTH = 8

    def fire(j):
        pltpu.async_copy(ones_v,
                         acc_sh.at[dst_all.at[pl.ds(loff + j * K, K)]],
                         ssem, add=True)

    def wait_one(j):
        pltpu.make_async_copy(ones_v,
                              acc_sh.at[dst_all.at[pl.ds(loff + j * K, K)]],
                              ssem).wait()

    def prol(j, _):
        fire(j)
        return 0

    def steady(j, _):
        wait_one(j - CDEPTH)
        fire(j)
        return 0

    def drain(j, _):
        wait_one(j)
        return 0

    ndeep = jnp.minimum(nb, CDEPTH)
    lax.fori_loop(0, ndeep, prol, 0)
    lax.fori_loop(CDEPTH, nb, steady, 0)
    lax.fori_loop(nb - ndeep, nb, drain, 0)
    plsc.subcore_barrier()
    pltpu.sync_copy(acc_sh.at[pl.ds(sid * RPS, RPS)],
                    out_hbm.at[cid, pl.ds(sid * RPS, RPS)])


KA = 32
NBA = EPW // KA
ABUF = 5


@functools.partial(
    pl.kernel,
    out_type=jax.ShapeDtypeStruct((NC, N_PAD, D), jnp.float32),
    mesh=_mesh,
    scratch_types=[
        pltpu.VMEM((EPW,), jnp.int32),
        pltpu.VMEM((EPW,), jnp.int32),
        pltpu.VMEM((ABUF, KA, D), jnp.float32),
        pltpu.SemaphoreType.DMA((ABUF,)),
        pltpu.VMEM_SHARED((N_PAD, D), jnp.float32),
    ],
)
def _sc_agg(h_hbm, src_hbm, dst_hbm, zeros_hbm, out_hbm,
            src_all, dst_all, rows_v, gsem, acc_sh):
    cid = lax.axis_index("c")
    sid = lax.axis_index("s")
    wid = sid * NC + cid
    base_st, loff, nba = _worker_range(wid, KA, NBA)
    pltpu.sync_copy(zeros_hbm.at[pl.ds(sid * RPS, RPS)],
                    acc_sh.at[pl.ds(sid * RPS, RPS)])
    pltpu.sync_copy(src_hbm.at[pl.ds(base_st, EPW)], src_all)
    pltpu.sync_copy(dst_hbm.at[pl.ds(base_st, EPW)], dst_all)
    plsc.subcore_barrier()

    def fire_gather(j, b):
        pltpu.async_copy(
            h_hbm.at[src_all.at[pl.ds(loff + j * KA, KA)]],
            rows_v.at[b], gsem.at[b])

    def drain_and_scatter(j, b):
        pltpu.make_async_copy(
            h_hbm.at[src_all.at[pl.ds(loff + j * KA, KA)]],
            rows_v.at[b], gsem.at[b]).wait()
        pltpu.sync_copy(rows_v.at[b],
                        acc_sh.at[dst_all.at[pl.ds(loff + j * KA, KA)]],
                        add=True)

    for b in range(ABUF):
        fire_gather(b, b)

    def body(g, _):
        for b in range(ABUF):
            j = g * ABUF + b
            drain_and_scatter(j, b)
            fire_gather(j + ABUF, b)
        return 0

    lax.fori_loop(0, nba // ABUF - 1, body, 0)
    for b in range(ABUF):
        drain_and_scatter(nba - ABUF + b, b)
    plsc.subcore_barrier()
    pltpu.sync_copy(acc_sh.at[pl.ds(sid * RPS, RPS)],
                    out_hbm.at[cid, pl.ds(sid * RPS, RPS)])


BN = 2000
GRID = N // BN

_row2 = lambda g: (g, 0)
_row3 = lambda g: (0, g, 0)
_full = lambda g: (0, 0)


def _t12_body(x_ref, w_ref, cnt_ref, hs_ref, dinv_ref):
    dinv = lax.rsqrt(1.0 + cnt_ref[...])
    hw = jnp.dot(x_ref[...], w_ref[...], preferred_element_type=jnp.float32)
    hs_ref[...] = hw * dinv
    dinv_ref[...] = dinv


def _tc_stage1(x, W1, cnt_col):
    return pl.pallas_call(
        _t12_body,
        grid=(GRID,),
        in_specs=[
            pl.BlockSpec((BN, D), _row2),
            pl.BlockSpec((D, D), _full),
            pl.BlockSpec((BN, 1), _row2),
        ],
        out_specs=[
            pl.BlockSpec((BN, D), _row2),
            pl.BlockSpec((BN, 1), _row2),
        ],
        out_shape=[
            jax.ShapeDtypeStruct((N, D), jnp.float32),
            jax.ShapeDtypeStruct((N, 1), jnp.float32),
        ],
    )(x, W1, cnt_col)


def _tmid_body(agg_ref, hs_ref, dinv_ref, b_ref, w_ref, out_ref):
    dinv = dinv_ref[...]
    h = dinv * (agg_ref[0] + agg_ref[1] + hs_ref[...]) + b_ref[...]
    h = jnp.maximum(h, 0.0)
    hw = jnp.dot(h, w_ref[...], preferred_element_type=jnp.float32)
    out_ref[...] = hw * dinv


def _tc_mid(agg, hs, dinv_col, b_row, W):
    return pl.pallas_call(
        _tmid_body,
        grid=(GRID,),
        in_specs=[
            pl.BlockSpec((NC, BN, D), _row3),
            pl.BlockSpec((BN, D), _row2),
            pl.BlockSpec((BN, 1), _row2),
            pl.BlockSpec((1, D), _full),
            pl.BlockSpec((D, D), _full),
        ],
        out_specs=pl.BlockSpec((BN, D), _row2),
        out_shape=jax.ShapeDtypeStruct((N, D), jnp.float32),
    )(agg, hs, dinv_col, b_row, W)


def _t4_body(agg_ref, hs_ref, dinv_ref, b_ref, out_ref):
    dinv = dinv_ref[...]
    h = dinv * (agg_ref[0] + agg_ref[1] + hs_ref[...]) + b_ref[...]
    out_ref[...] = dinv * jnp.maximum(h, 0.0)


def _tc_pre3(agg, hs2, dinv_col, b2_row):
    return pl.pallas_call(
        _t4_body,
        grid=(GRID,),
        in_specs=[
            pl.BlockSpec((NC, BN, D), _row3),
            pl.BlockSpec((BN, D), _row2),
            pl.BlockSpec((BN, 1), _row2),
            pl.BlockSpec((1, D), _full),
        ],
        out_specs=pl.BlockSpec((BN, D), _row2),
        out_shape=jax.ShapeDtypeStruct((N, D), jnp.float32),
    )(agg, hs2, dinv_col, b2_row)


def _t5_body(agg_ref, g_ref, dinv_ref, w_ref, b_ref, out_ref):
    z = dinv_ref[...] * (agg_ref[0] + agg_ref[1] + g_ref[...])
    out_ref[...] = (jnp.dot(z, w_ref[...], preferred_element_type=jnp.float32)
                    + b_ref[...])


def _tc_final(agg, g, dinv_col, W3p, b3_row):
    return pl.pallas_call(
        _t5_body,
        grid=(GRID,),
        in_specs=[
            pl.BlockSpec((NC, BN, D), _row3),
            pl.BlockSpec((BN, D), _row2),
            pl.BlockSpec((BN, 1), _row2),
            pl.BlockSpec((D, DP3), _full),
            pl.BlockSpec((1, DP3), _full),
        ],
        out_specs=pl.BlockSpec((BN, DP3), _row2),
        out_shape=jax.ShapeDtypeStruct((N, DP3), jnp.float32),
    )(agg, g, dinv_col, W3p, b3_row)


def kernel(x, edge_index, W1, b1, W2, b2, W3, b3):
    W3p = jnp.pad(W3, ((0, 0), (0, DP3 - W3.shape[1])))
    b1r = b1.reshape(1, D)
    b2r = b2.reshape(1, D)
    b3r = jnp.pad(b3, (0, DP3 - b3.shape[0])).reshape(1, DP3)
    z1 = jnp.zeros((N_PAD,), jnp.float32)
    z128 = jnp.zeros((N_PAD, D), jnp.float32)

    src = edge_index[0]
    dst = edge_index[1]

    cnt_parts = _sc_count(dst, z1)
    cnt_col = (cnt_parts[0] + cnt_parts[1])[:N].reshape(N, 1)

    hs1, dinv_col = _tc_stage1(x, W1, cnt_col)

    agg1 = _sc_agg(hs1, src, dst, z128)
    hs2 = _tc_mid(agg1, hs1, dinv_col, b1r, W2)

    agg2 = _sc_agg(hs2, src, dst, z128)
    g = _tc_pre3(agg2, hs2, dinv_col, b2r)

    agg3 = _sc_agg(g, src, dst, z128)
    out16 = _tc_final(agg3, g, dinv_col, W3p, b3r)

    return out16[:, :2]
